# Initial kernel scaffold; baseline (speedup 1.0000x reference)
import jax
import jax.numpy as jnp
from jax.experimental import pallas as pl


def kernel(x, edge_index, edge_weight, W1, b1, W2, b2, W3, b3, fcW, fcb):
    raise NotImplementedError("write your pallas kernel here")


# speedup vs baseline: 2.5082x; 2.5082x over previous
import functools

import jax
import jax.numpy as jnp
from jax import lax
from jax.experimental import pallas as pl
from jax.experimental.pallas import tpu as pltpu
from jax.experimental.pallas import tpu_sc as plsc

NCORES = 2
NTILES = 16
LANES = 16


def _make_spmm(n, e, ngrp, gw, k):
    ept = e // NTILES
    assert ept * NTILES == e and ept % k == 0
    ch = ept // k
    rows_pt = n // NTILES
    assert rows_pt * NTILES == n and rows_pt % 8 == 0
    zr = rows_pt
    while zr * gw * 4 > 256 * 1024:
        zr //= 2
    assert rows_pt % zr == 0 and zr % 8 == 0
    mesh = plsc.VectorSubcoreMesh(core_axis_name="c", subcore_axis_name="s")

    @functools.partial(
        pl.kernel,
        out_type=jax.ShapeDtypeStruct((NCORES, ngrp, n, gw), jnp.float32),
        mesh=mesh,
        scratch_types=[
            pltpu.VMEM((ch, k), jnp.int32),
            pltpu.VMEM((ch, k), jnp.int32),
            pltpu.VMEM((ch, k), jnp.float32),
            pltpu.VMEM((k, gw), jnp.float32),
            pltpu.VMEM((zr, gw), jnp.float32),
            pltpu.VMEM_SHARED((n, gw), jnp.float32),
        ],
        compiler_params=pltpu.CompilerParams(use_tc_tiling_on_sc=False),
    )
    def spmm(h_hbm, src_hbm, dst_hbm, w_hbm, out_hbm,
             src_v, dst_v, w_v, rows_v, zero_v, acc_sh):
        c = lax.axis_index("c")
        s = lax.axis_index("s")
        pltpu.sync_copy(src_hbm.at[s], src_v)
        pltpu.sync_copy(dst_hbm.at[s], dst_v)
        pltpu.sync_copy(w_hbm.at[s], w_v)

        @pl.loop(0, zr)
        def _(r):
            for v in range(gw // LANES):
                zero_v[r, pl.ds(v * LANES, LANES)] = jnp.zeros((LANES,), jnp.float32)

        for g in range(ngrp):
            @pl.loop(0, rows_pt // zr)
            def _(j):
                pltpu.sync_copy(zero_v, acc_sh.at[pl.ds(s * rows_pt + j * zr, zr)])

            plsc.subcore_barrier()

            @pl.loop(0, ch)
            def _(ci):
                pltpu.sync_copy(h_hbm.at[c, g].at[src_v.at[ci]], rows_v)
                for grp in range(k // LANES):
                    w16 = w_v[ci, pl.ds(grp * LANES, LANES)]
                    for i in range(LANES):
                        ei = grp * LANES + i
                        wgt = w16[i]
                        for v in range(gw // LANES):
                            sl = (ei, pl.ds(v * LANES, LANES))
                            rows_v[sl] = rows_v[sl] * wgt
                pltpu.sync_copy(rows_v, acc_sh.at[dst_v.at[ci]], add=True)

            plsc.subcore_barrier()
            pltpu.sync_copy(acc_sh.at[pl.ds(s * rows_pt, rows_pt)],
                            out_hbm.at[c, g, pl.ds(s * rows_pt, rows_pt)])
            plsc.subcore_barrier()

    return spmm


def _col_blocks(ngrp, gw):
    return [(c, g, (c * ngrp + g) * gw) for c in range(2) for g in range(ngrp)]


_ROWB = 1280


def _mm_in(x, w, ngrp):
    n, f = x.shape
    gw = w.shape[1] // (2 * ngrp)

    def body(x_ref, w_ref, o_ref):
        xx = x_ref[...]
        for c, g, lo in _col_blocks(ngrp, gw):
            o_ref[c, g, :, :] = jnp.dot(xx, w_ref[:, lo:lo + gw],
                                        preferred_element_type=jnp.float32,
                                        precision=lax.Precision.HIGHEST)

    return pl.pallas_call(
        body,
        grid=(n // _ROWB,),
        in_specs=[pl.BlockSpec((_ROWB, f), lambda i: (i, 0)),
                  pl.BlockSpec(w.shape, lambda i: (0, 0))],
        out_specs=pl.BlockSpec((2, ngrp, _ROWB, gw), lambda i: (0, 0, i, 0)),
        out_shape=jax.ShapeDtypeStruct((2, ngrp, n, gw), jnp.float32),
    )(x, w)


def _mm_mid(a, b, w, ngrp_out):
    _, ngrp_in, n, gw_in = a.shape
    gw_out = w.shape[1] // (2 * ngrp_out)
    b4 = b.reshape(2, ngrp_in, 1, gw_in)

    def body(a_ref, b_ref, w_ref, o_ref):
        h = jnp.concatenate(
            [jnp.maximum(a_ref[c, g] + b_ref[c, g], 0.0)
             for c in range(2) for g in range(ngrp_in)], axis=1)
        for c, g, lo in _col_blocks(ngrp_out, gw_out):
            o_ref[c, g, :, :] = jnp.dot(h, w_ref[:, lo:lo + gw_out],
                                        preferred_element_type=jnp.float32,
                                        precision=lax.Precision.HIGHEST)

    return pl.pallas_call(
        body,
        grid=(n // _ROWB,),
        in_specs=[pl.BlockSpec((2, ngrp_in, _ROWB, gw_in),
                               lambda i: (0, 0, i, 0)),
                  pl.BlockSpec(b4.shape, lambda i: (0, 0, 0, 0)),
                  pl.BlockSpec(w.shape, lambda i: (0, 0))],
        out_specs=pl.BlockSpec((2, ngrp_out, _ROWB, gw_out),
                               lambda i: (0, 0, i, 0)),
        out_shape=jax.ShapeDtypeStruct((2, ngrp_out, n, gw_out), jnp.float32),
    )(a, b4, w)


def _mm_fc(a, b, w, fcb):
    _, ngrp_in, n, gw_in = a.shape
    ncols = w.shape[1]
    b4 = b.reshape(2, ngrp_in, 1, gw_in)

    def body(a_ref, b_ref, w_ref, fcb_ref, o_ref):
        h = jnp.concatenate(
            [jnp.maximum(a_ref[c, g] + b_ref[c, g], 0.0)
             for c in range(2) for g in range(ngrp_in)], axis=1)
        o_ref[...] = jnp.dot(h, w_ref[...],
                             preferred_element_type=jnp.float32,
                             precision=lax.Precision.HIGHEST) + fcb_ref[...]

    return pl.pallas_call(
        body,
        grid=(n // _ROWB,),
        in_specs=[pl.BlockSpec((2, ngrp_in, _ROWB, gw_in),
                               lambda i: (0, 0, i, 0)),
                  pl.BlockSpec(b4.shape, lambda i: (0, 0, 0, 0)),
                  pl.BlockSpec(w.shape, lambda i: (0, 0)),
                  pl.BlockSpec(fcb.shape, lambda i: (0,))],
        out_specs=pl.BlockSpec((_ROWB, ncols), lambda i: (i, 0)),
        out_shape=jax.ShapeDtypeStruct((n, ncols), jnp.float32),
    )(a, b4, w, fcb)


def _pad2(m, rows, cols):
    return jnp.pad(m, ((0, rows - m.shape[0]), (0, cols - m.shape[1])))


def kernel(x, edge_index, edge_weight, W1, b1, W2, b2, W3, b3, fcW, fcb):
    n, _ = x.shape
    e = edge_index.shape[1]
    h1p, h2p = 384, 96
    g1, gw1 = 3, 64
    g2, gw2 = 1, 48
    k = 80
    ch = e // NTILES // k
    npad = ((n + NTILES * 16 - 1) // (NTILES * 16)) * (NTILES * 16)
    x = jnp.pad(x, ((0, npad - n), (0, 0)))

    src = edge_index[0].astype(jnp.int32).reshape(NTILES, ch, k)
    dst = edge_index[1].astype(jnp.int32).reshape(NTILES, ch, k)
    w = edge_weight.reshape(NTILES, ch, k)

    W1p = _pad2(W1, W1.shape[0], h1p)
    b1p = jnp.pad(b1, (0, h1p - b1.shape[0]))
    W2p = _pad2(W2, h1p, h2p)
    b2p = jnp.pad(b2, (0, h2p - b2.shape[0]))
    W3p = _pad2(W3, h2p, h2p)
    b3p = jnp.pad(b3, (0, h2p - b3.shape[0]))
    ncls = fcW.shape[1]
    fcWp = _pad2(fcW, h2p, 128)
    fcbp = jnp.pad(fcb, (0, 128 - ncls))

    spmm1 = _make_spmm(npad, e, g1, gw1, k)
    spmm23 = _make_spmm(npad, e, g2, gw2, k)

    h = _mm_in(x, W1p, g1)
    h = spmm1(h, src, dst, w)
    h = _mm_mid(h, b1p, W2p, g2)
    h = spmm23(h, src, dst, w)
    h = _mm_mid(h, b2p, W3p, g2)
    h = spmm23(h, src, dst, w)
    out = _mm_fc(h, b3p, fcWp, fcbp)
    return out[:n, :ncls]


# speedup vs baseline: 4.1753x; 1.6647x over previous
import functools

import jax
import jax.numpy as jnp
from jax import lax
from jax.experimental import pallas as pl
from jax.experimental.pallas import tpu as pltpu
from jax.experimental.pallas import tpu_sc as plsc

NCORES = 2
NTILES = 16
LANES = 16


def _make_spmm(n, e, ngrp, gw, k):
    ept = e // NTILES
    assert ept * NTILES == e and ept % k == 0
    ch = ept // k
    rows_pt = n // NTILES
    assert rows_pt * NTILES == n and rows_pt % 8 == 0
    zr = rows_pt
    while zr * gw * 4 > 128 * 1024:
        zr //= 2
    assert rows_pt % zr == 0 and zr % 8 == 0
    mesh = plsc.VectorSubcoreMesh(core_axis_name="c", subcore_axis_name="s")

    nb = 5
    assert ch % nb == 0

    @functools.partial(
        pl.kernel,
        out_type=jax.ShapeDtypeStruct((NCORES, ngrp, n, gw), jnp.float32),
        mesh=mesh,
        scratch_types=[
            pltpu.VMEM((ch, k), jnp.int32),
            pltpu.VMEM((ch, k), jnp.int32),
            pltpu.VMEM((ch, k), jnp.float32),
            pltpu.VMEM((nb, k, gw), jnp.float32),
            pltpu.VMEM((zr, gw), jnp.float32),
            pltpu.VMEM_SHARED((n, gw), jnp.float32),
            pltpu.SemaphoreType.DMA((nb,)),
            pltpu.SemaphoreType.DMA((nb,)),
        ],
        compiler_params=pltpu.CompilerParams(use_tc_tiling_on_sc=False),
    )
    def spmm(h_hbm, src_hbm, dst_hbm, w_hbm, out_hbm,
             src_v, dst_v, w_v, rows_v, zero_v, acc_sh, sem_g, sem_s):
        c = lax.axis_index("c")
        s = lax.axis_index("s")
        pltpu.sync_copy(src_hbm.at[s], src_v)
        pltpu.sync_copy(dst_hbm.at[s], dst_v)
        pltpu.sync_copy(w_hbm.at[s], w_v)

        @pl.loop(0, zr)
        def _(r):
            for v in range(gw // LANES):
                zero_v[r, pl.ds(v * LANES, LANES)] = jnp.zeros((LANES,), jnp.float32)

        def gather(g, ci, b):
            pltpu.async_copy(h_hbm.at[c, g].at[src_v.at[ci]],
                             rows_v.at[b], sem_g.at[b])

        @pl.loop(0, ngrp)
        def _(g):
            @pl.loop(0, rows_pt // zr)
            def _(j):
                pltpu.sync_copy(zero_v, acc_sh.at[pl.ds(s * rows_pt + j * zr, zr)])

            plsc.subcore_barrier()

            for b in range(nb - 1):
                gather(g, b, b)

            @pl.loop(0, ch // nb)
            def _(t):
                for j in range(nb):
                    ci = t * nb + j
                    pltpu.make_async_copy(h_hbm.at[c, g].at[src_v.at[ci]],
                                          rows_v.at[j], sem_g.at[j]).wait()
                    for grp in range(k // LANES):
                        w16 = w_v[ci, pl.ds(grp * LANES, LANES)]
                        for i in range(LANES):
                            ei = grp * LANES + i
                            wgt = w16[i]
                            for v in range(gw // LANES):
                                sl = (j, ei, pl.ds(v * LANES, LANES))
                                rows_v[sl] = rows_v[sl] * wgt
                    jp = (j - 1) % nb
                    prev_sc = pltpu.make_async_copy(
                        rows_v.at[jp], acc_sh.at[dst_v.at[ci - 1]],
                        sem_s.at[jp])
                    if j == 0:
                        @pl.when(t > 0)
                        def _():
                            prev_sc.wait()
                    else:
                        prev_sc.wait()
                    pltpu.async_copy(rows_v.at[j], acc_sh.at[dst_v.at[ci]],
                                     sem_s.at[j], add=True)

                    @pl.when(ci + nb - 1 < ch)
                    def _():
                        gather(g, ci + nb - 1, jp)

            pltpu.make_async_copy(rows_v.at[nb - 1],
                                  acc_sh.at[dst_v.at[ch - 1]],
                                  sem_s.at[nb - 1]).wait()
            plsc.subcore_barrier()
            pltpu.sync_copy(acc_sh.at[pl.ds(s * rows_pt, rows_pt)],
                            out_hbm.at[c, g, pl.ds(s * rows_pt, rows_pt)])
            plsc.subcore_barrier()

    return spmm


def _col_blocks(ngrp, gw):
    return [(c, g, (c * ngrp + g) * gw) for c in range(2) for g in range(ngrp)]


_ROWB = 1280


def _mm_in(x, w, ngrp):
    n, f = x.shape
    gw = w.shape[1] // (2 * ngrp)

    def body(x_ref, w_ref, o_ref):
        xx = x_ref[...]
        for c, g, lo in _col_blocks(ngrp, gw):
            o_ref[c, g, :, :] = jnp.dot(xx, w_ref[:, lo:lo + gw],
                                        preferred_element_type=jnp.float32,
                                        precision=lax.Precision.HIGHEST)

    return pl.pallas_call(
        body,
        grid=(n // _ROWB,),
        in_specs=[pl.BlockSpec((_ROWB, f), lambda i: (i, 0)),
                  pl.BlockSpec(w.shape, lambda i: (0, 0))],
        out_specs=pl.BlockSpec((2, ngrp, _ROWB, gw), lambda i: (0, 0, i, 0)),
        out_shape=jax.ShapeDtypeStruct((2, ngrp, n, gw), jnp.float32),
    )(x, w)


def _mm_mid(a, b, w, ngrp_out):
    _, ngrp_in, n, gw_in = a.shape
    gw_out = w.shape[1] // (2 * ngrp_out)
    b4 = b.reshape(2, ngrp_in, 1, gw_in)

    def body(a_ref, b_ref, w_ref, o_ref):
        h = jnp.concatenate(
            [jnp.maximum(a_ref[c, g] + b_ref[c, g], 0.0)
             for c in range(2) for g in range(ngrp_in)], axis=1)
        for c, g, lo in _col_blocks(ngrp_out, gw_out):
            o_ref[c, g, :, :] = jnp.dot(h, w_ref[:, lo:lo + gw_out],
                                        preferred_element_type=jnp.float32,
                                        precision=lax.Precision.HIGHEST)

    return pl.pallas_call(
        body,
        grid=(n // _ROWB,),
        in_specs=[pl.BlockSpec((2, ngrp_in, _ROWB, gw_in),
                               lambda i: (0, 0, i, 0)),
                  pl.BlockSpec(b4.shape, lambda i: (0, 0, 0, 0)),
                  pl.BlockSpec(w.shape, lambda i: (0, 0))],
        out_specs=pl.BlockSpec((2, ngrp_out, _ROWB, gw_out),
                               lambda i: (0, 0, i, 0)),
        out_shape=jax.ShapeDtypeStruct((2, ngrp_out, n, gw_out), jnp.float32),
    )(a, b4, w)


def _mm_fc(a, b, w, fcb):
    _, ngrp_in, n, gw_in = a.shape
    ncols = w.shape[1]
    b4 = b.reshape(2, ngrp_in, 1, gw_in)

    def body(a_ref, b_ref, w_ref, fcb_ref, o_ref):
        h = jnp.concatenate(
            [jnp.maximum(a_ref[c, g] + b_ref[c, g], 0.0)
             for c in range(2) for g in range(ngrp_in)], axis=1)
        o_ref[...] = jnp.dot(h, w_ref[...],
                             preferred_element_type=jnp.float32,
                             precision=lax.Precision.HIGHEST) + fcb_ref[...]

    return pl.pallas_call(
        body,
        grid=(n // _ROWB,),
        in_specs=[pl.BlockSpec((2, ngrp_in, _ROWB, gw_in),
                               lambda i: (0, 0, i, 0)),
                  pl.BlockSpec(b4.shape, lambda i: (0, 0, 0, 0)),
                  pl.BlockSpec(w.shape, lambda i: (0, 0)),
                  pl.BlockSpec(fcb.shape, lambda i: (0,))],
        out_specs=pl.BlockSpec((_ROWB, ncols), lambda i: (i, 0)),
        out_shape=jax.ShapeDtypeStruct((n, ncols), jnp.float32),
    )(a, b4, w, fcb)


def _pad2(m, rows, cols):
    return jnp.pad(m, ((0, rows - m.shape[0]), (0, cols - m.shape[1])))


def kernel(x, edge_index, edge_weight, W1, b1, W2, b2, W3, b3, fcW, fcb):
    n, _ = x.shape
    e = edge_index.shape[1]
    h1p, h2p = 384, 96
    g1, gw1 = 3, 64
    g2, gw2 = 1, 48
    k = 80
    ch = e // NTILES // k
    npad = ((n + NTILES * 16 - 1) // (NTILES * 16)) * (NTILES * 16)
    x = jnp.pad(x, ((0, npad - n), (0, 0)))

    src = edge_index[0].astype(jnp.int32).reshape(NTILES, ch, k)
    dst = edge_index[1].astype(jnp.int32).reshape(NTILES, ch, k)
    w = edge_weight.reshape(NTILES, ch, k)

    W1p = _pad2(W1, W1.shape[0], h1p)
    b1p = jnp.pad(b1, (0, h1p - b1.shape[0]))
    W2p = _pad2(W2, h1p, h2p)
    b2p = jnp.pad(b2, (0, h2p - b2.shape[0]))
    W3p = _pad2(W3, h2p, h2p)
    b3p = jnp.pad(b3, (0, h2p - b3.shape[0]))
    ncls = fcW.shape[1]
    fcWp = _pad2(fcW, h2p, 128)
    fcbp = jnp.pad(fcb, (0, 128 - ncls))

    spmm1 = _make_spmm(npad, e, g1, gw1, k)
    spmm23 = _make_spmm(npad, e, g2, gw2, k)

    h = _mm_in(x, W1p, g1)
    h = spmm1(h, src, dst, w)
    h = _mm_mid(h, b1p, W2p, g2)
    h = spmm23(h, src, dst, w)
    h = _mm_mid(h, b2p, W3p, g2)
    h = spmm23(h, src, dst, w)
    out = _mm_fc(h, b3p, fcWp, fcbp)
    return out[:n, :ncls]


# speedup vs baseline: 4.7033x; 1.1264x over previous
import functools

import jax
import jax.numpy as jnp
from jax import lax
from jax.experimental import pallas as pl
from jax.experimental.pallas import tpu as pltpu
from jax.experimental.pallas import tpu_sc as plsc

NCORES = 2
NTILES = 16
LANES = 16
LEAD = 3
_MM_PREC = lax.Precision.DEFAULT


def _make_spmm(n, e, ngrp, gw, k):
    ept = e // NTILES
    assert ept * NTILES == e and ept % k == 0
    ch = ept // k
    rows_pt = n // NTILES
    assert rows_pt * NTILES == n and rows_pt % 8 == 0
    zr = rows_pt
    while zr * gw * 4 > 128 * 1024:
        zr //= 2
    assert rows_pt % zr == 0 and zr % 8 == 0
    mesh = plsc.VectorSubcoreMesh(core_axis_name="c", subcore_axis_name="s")

    nb = 5
    assert ch % nb == 0

    @functools.partial(
        pl.kernel,
        out_type=jax.ShapeDtypeStruct((NCORES, ngrp, n, gw), jnp.float32),
        mesh=mesh,
        scratch_types=[
            pltpu.VMEM((ch, k), jnp.int32),
            pltpu.VMEM((ch, k), jnp.int32),
            pltpu.VMEM((ch, k), jnp.float32),
            pltpu.VMEM((nb, k, gw), jnp.float32),
            pltpu.VMEM((zr, gw), jnp.float32),
            pltpu.VMEM_SHARED((n, gw), jnp.float32),
            pltpu.SemaphoreType.DMA((nb,)),
            pltpu.SemaphoreType.DMA((nb,)),
        ],
        compiler_params=pltpu.CompilerParams(use_tc_tiling_on_sc=False),
    )
    def spmm(h_hbm, src_hbm, dst_hbm, w_hbm, out_hbm,
             src_v, dst_v, w_v, rows_v, zero_v, acc_sh, sem_g, sem_s):
        c = lax.axis_index("c")
        s = lax.axis_index("s")
        pltpu.sync_copy(src_hbm.at[s], src_v)
        pltpu.sync_copy(dst_hbm.at[s], dst_v)
        pltpu.sync_copy(w_hbm.at[s], w_v)

        @pl.loop(0, zr)
        def _(r):
            for v in range(gw // LANES):
                zero_v[r, pl.ds(v * LANES, LANES)] = jnp.zeros((LANES,), jnp.float32)

        def gather(g, ci, b):
            pltpu.async_copy(h_hbm.at[c, g].at[src_v.at[ci]],
                             rows_v.at[b], sem_g.at[b])

        @pl.loop(0, ngrp)
        def _(g):
            @pl.loop(0, rows_pt // zr)
            def _(j):
                pltpu.sync_copy(zero_v, acc_sh.at[pl.ds(s * rows_pt + j * zr, zr)])

            plsc.subcore_barrier()

            for b in range(LEAD):
                gather(g, b, b)

            @pl.loop(0, ch // nb)
            def _(t):
                for j in range(nb):
                    ci = t * nb + j
                    pltpu.make_async_copy(h_hbm.at[c, g].at[src_v.at[ci]],
                                          rows_v.at[j], sem_g.at[j]).wait()
                    for grp in range(k // LANES):
                        w16 = w_v[ci, pl.ds(grp * LANES, LANES)]
                        for i in range(LANES):
                            ei = grp * LANES + i
                            wgt = w16[i]
                            for v in range(gw // LANES):
                                sl = (j, ei, pl.ds(v * LANES, LANES))
                                rows_v[sl] = rows_v[sl] * wgt
                    pltpu.async_copy(rows_v.at[j], acc_sh.at[dst_v.at[ci]],
                                     sem_s.at[j], add=True)
                    sp = (j + LEAD) % nb
                    prev_sc = pltpu.make_async_copy(
                        rows_v.at[sp], acc_sh.at[dst_v.at[ci - (nb - LEAD)]],
                        sem_s.at[sp])
                    if j < nb - LEAD:
                        @pl.when(t > 0)
                        def _():
                            prev_sc.wait()
                    else:
                        prev_sc.wait()

                    @pl.when(ci + LEAD < ch)
                    def _():
                        gather(g, ci + LEAD, sp)

            for ci in range(ch - (nb - LEAD), ch):
                pltpu.make_async_copy(rows_v.at[ci % nb],
                                      acc_sh.at[dst_v.at[ci]],
                                      sem_s.at[ci % nb]).wait()
            plsc.subcore_barrier()
            pltpu.sync_copy(acc_sh.at[pl.ds(s * rows_pt, rows_pt)],
                            out_hbm.at[c, g, pl.ds(s * rows_pt, rows_pt)])
            plsc.subcore_barrier()

    return spmm


def _col_blocks(ngrp, gw):
    return [(c, g, (c * ngrp + g) * gw) for c in range(2) for g in range(ngrp)]


_ROWB = 1280


def _mm_in(x, w, ngrp):
    n, f = x.shape
    gw = w.shape[1] // (2 * ngrp)

    def body(x_ref, w_ref, o_ref):
        xx = x_ref[...]
        for c, g, lo in _col_blocks(ngrp, gw):
            o_ref[c, g, :, :] = jnp.dot(xx, w_ref[:, lo:lo + gw],
                                        preferred_element_type=jnp.float32,
                                        precision=_MM_PREC)

    return pl.pallas_call(
        body,
        grid=(n // _ROWB,),
        in_specs=[pl.BlockSpec((_ROWB, f), lambda i: (i, 0)),
                  pl.BlockSpec(w.shape, lambda i: (0, 0))],
        out_specs=pl.BlockSpec((2, ngrp, _ROWB, gw), lambda i: (0, 0, i, 0)),
        out_shape=jax.ShapeDtypeStruct((2, ngrp, n, gw), jnp.float32),
    )(x, w)


def _mm_mid(a, b, w, ngrp_out):
    _, ngrp_in, n, gw_in = a.shape
    gw_out = w.shape[1] // (2 * ngrp_out)
    b4 = b.reshape(2, ngrp_in, 1, gw_in)

    def body(a_ref, b_ref, w_ref, o_ref):
        h = jnp.concatenate(
            [jnp.maximum(a_ref[c, g] + b_ref[c, g], 0.0)
             for c in range(2) for g in range(ngrp_in)], axis=1)
        for c, g, lo in _col_blocks(ngrp_out, gw_out):
            o_ref[c, g, :, :] = jnp.dot(h, w_ref[:, lo:lo + gw_out],
                                        preferred_element_type=jnp.float32,
                                        precision=_MM_PREC)

    return pl.pallas_call(
        body,
        grid=(n // _ROWB,),
        in_specs=[pl.BlockSpec((2, ngrp_in, _ROWB, gw_in),
                               lambda i: (0, 0, i, 0)),
                  pl.BlockSpec(b4.shape, lambda i: (0, 0, 0, 0)),
                  pl.BlockSpec(w.shape, lambda i: (0, 0))],
        out_specs=pl.BlockSpec((2, ngrp_out, _ROWB, gw_out),
                               lambda i: (0, 0, i, 0)),
        out_shape=jax.ShapeDtypeStruct((2, ngrp_out, n, gw_out), jnp.float32),
    )(a, b4, w)


def _mm_fc(a, b, w, fcb):
    _, ngrp_in, n, gw_in = a.shape
    ncols = w.shape[1]
    b4 = b.reshape(2, ngrp_in, 1, gw_in)

    def body(a_ref, b_ref, w_ref, fcb_ref, o_ref):
        h = jnp.concatenate(
            [jnp.maximum(a_ref[c, g] + b_ref[c, g], 0.0)
             for c in range(2) for g in range(ngrp_in)], axis=1)
        o_ref[...] = jnp.dot(h, w_ref[...],
                             preferred_element_type=jnp.float32,
                             precision=_MM_PREC) + fcb_ref[...]

    return pl.pallas_call(
        body,
        grid=(n // _ROWB,),
        in_specs=[pl.BlockSpec((2, ngrp_in, _ROWB, gw_in),
                               lambda i: (0, 0, i, 0)),
                  pl.BlockSpec(b4.shape, lambda i: (0, 0, 0, 0)),
                  pl.BlockSpec(w.shape, lambda i: (0, 0)),
                  pl.BlockSpec(fcb.shape, lambda i: (0,))],
        out_specs=pl.BlockSpec((_ROWB, ncols), lambda i: (i, 0)),
        out_shape=jax.ShapeDtypeStruct((n, ncols), jnp.float32),
    )(a, b4, w, fcb)


def _pad2(m, rows, cols):
    return jnp.pad(m, ((0, rows - m.shape[0]), (0, cols - m.shape[1])))


def kernel(x, edge_index, edge_weight, W1, b1, W2, b2, W3, b3, fcW, fcb):
    n, _ = x.shape
    e = edge_index.shape[1]
    h1p, h2p = 384, 96
    g1, gw1 = 3, 64
    g2, gw2 = 1, 48
    k = 80
    ch = e // NTILES // k
    npad = ((n + NTILES * 16 - 1) // (NTILES * 16)) * (NTILES * 16)
    x = jnp.pad(x, ((0, npad - n), (0, 0)))

    src = edge_index[0].astype(jnp.int32).reshape(NTILES, ch, k)
    dst = edge_index[1].astype(jnp.int32).reshape(NTILES, ch, k)
    w = edge_weight.reshape(NTILES, ch, k)

    W1p = _pad2(W1, W1.shape[0], h1p)
    b1p = jnp.pad(b1, (0, h1p - b1.shape[0]))
    W2p = _pad2(W2, h1p, h2p)
    b2p = jnp.pad(b2, (0, h2p - b2.shape[0]))
    W3p = _pad2(W3, h2p, h2p)
    b3p = jnp.pad(b3, (0, h2p - b3.shape[0]))
    ncls = fcW.shape[1]
    fcWp = _pad2(fcW, h2p, 128)
    fcbp = jnp.pad(fcb, (0, 128 - ncls))

    spmm1 = _make_spmm(npad, e, g1, gw1, k)
    spmm23 = _make_spmm(npad, e, g2, gw2, k)

    h = _mm_in(x, W1p, g1)
    h = spmm1(h, src, dst, w)
    h = _mm_mid(h, b1p, W2p, g2)
    h = spmm23(h, src, dst, w)
    h = _mm_mid(h, b2p, W3p, g2)
    h = spmm23(h, src, dst, w)
    out = _mm_fc(h, b3p, fcWp, fcbp)
    return out[:n, :ncls]
